# Initial kernel scaffold; baseline (speedup 1.0000x reference)
import jax
import jax.numpy as jnp
from jax.experimental import pallas as pl


def kernel(x, edge_index, batch, W1, a_src1, a_dst1, b1, W2, a_src2, a_dst2, b2, wa, ba, wm, bm, Wo, bo):
    raise NotImplementedError("write your pallas kernel here")


# speedup vs baseline: 1.1243x; 1.1243x over previous
import functools

import jax
import jax.numpy as jnp
from jax.experimental import pallas as pl
from jax.experimental.pallas import tpu as pltpu

N = 10000
E = 320000
FEAT = 128
HEADS = 8
C1 = 128
C2 = 128
OUT = 10
G = 64


def _matmul_kernel(x_ref, w_ref, o_ref):
    o_ref[...] = jnp.dot(x_ref[...], w_ref[...],
                         preferred_element_type=jnp.float32)


def _pallas_matmul(x, w):
    m, k = x.shape
    k2, n = w.shape
    return pl.pallas_call(
        _matmul_kernel,
        out_shape=jax.ShapeDtypeStruct((m, n), jnp.float32),
    )(x, w)


def _gat_layer(x, edge_index, W, a_src, a_dst, b, heads, ch):
    n = x.shape[0]
    h = (x @ W).reshape(n, heads, ch)
    src = edge_index[0]
    dst = edge_index[1]
    alpha_src = (h * a_src[None, :, :]).sum(-1)
    alpha_dst = (h * a_dst[None, :, :]).sum(-1)
    e = alpha_src[src] + alpha_dst[dst]
    e = jnp.where(e > 0, e, 0.2 * e)
    ex = jnp.exp(e)
    den = jax.ops.segment_sum(ex, dst, num_segments=n)
    num = jax.ops.segment_sum(h[src] * ex[:, :, None], dst, num_segments=n)
    out = num / (den[:, :, None] + 1e-16)
    return out.reshape(n, heads * ch) + b


def kernel(x, edge_index, batch, W1, a_src1, a_dst1, b1, W2, a_src2, a_dst2,
           b2, wa, ba, wm, bm, Wo, bo):
    h = _gat_layer(x, edge_index, W1, a_src1, a_dst1, b1, HEADS, C1)
    h = jax.nn.relu(h)
    h = _gat_layer(h, edge_index, W2, a_src2, a_dst2, b2, 1, C2)
    h = jax.nn.relu(h)
    attn_scores = h @ wa + ba
    node_mask = jax.nn.sigmoid(h @ wm + bm)
    final_scores = attn_scores * node_mask
    pooled = jax.ops.segment_sum(h * final_scores, batch, num_segments=G)
    return _pallas_matmul(pooled, Wo) + bo


# speedup vs baseline: 10.2151x; 9.0861x over previous
import functools

import jax
import jax.numpy as jnp
from jax import lax
from jax.experimental import pallas as pl
from jax.experimental.pallas import tpu as pltpu
from jax.experimental.pallas import tpu_sc as plsc

N = 10000
NP = 10240
E = 320000
EP = 327680
FEAT = 128
HEADS = 8
C1 = 128
C2 = 128
OUT = 10
G = 64
FA = 144
DUMMY = N

NBLK = NP // 256

NC = 2
NS = 16
CHUNK = 128

EP_T1 = EP // NS
NCH1 = EP_T1 // CHUNK
EP_T2 = EP // (NS * NC)
NCH2 = EP_T2 // CHUNK
ROWS_T = NP // NS


def _a_body(x_ref, w_ref, asrc_ref, adst_ref, haug_ref, ad_ref):
    h = jnp.dot(x_ref[...], w_ref[...], preferred_element_type=jnp.float32)
    lane = lax.broadcasted_iota(jnp.int32, (256, 16), 1)
    ones_col = (lane == 0).astype(jnp.float32)
    as_col = (lane == 1).astype(jnp.float32)
    for hh in range(HEADS):
        hblk = h[:, hh * C1:(hh + 1) * C1]
        a_s = jnp.sum(hblk * asrc_ref[hh, :][None, :], axis=1)
        haug_ref[hh, :, 0:C1] = hblk
        haug_ref[hh, :, C1:FA] = ones_col + as_col * a_s[:, None]
        ad_ref[hh, :] = jnp.sum(hblk * adst_ref[hh, :][None, :], axis=1)


def _layer1_dense(x_pad, W1, a_src1, a_dst1):
    return pl.pallas_call(
        _a_body,
        grid=(NBLK,),
        in_specs=[
            pl.BlockSpec((256, FEAT), lambda i: (i, 0)),
            pl.BlockSpec((FEAT, HEADS * C1), lambda i: (0, 0)),
            pl.BlockSpec((HEADS, C1), lambda i: (0, 0)),
            pl.BlockSpec((HEADS, C1), lambda i: (0, 0)),
        ],
        out_specs=[
            pl.BlockSpec((HEADS, 256, FA), lambda i: (0, i, 0)),
            pl.BlockSpec((HEADS, 256), lambda i: (0, i)),
        ],
        out_shape=[
            jax.ShapeDtypeStruct((HEADS, NP, FA), jnp.float32),
            jax.ShapeDtypeStruct((HEADS, NP), jnp.float32),
        ],
    )(x_pad, W1, a_src1, a_dst1)


GRP = 16


def _sc_edge_body(nheads, nch, split_cores, h_hbm, ad_hbm, src_hbm, dst_hbm,
                  zero_hbm, num_hbm, src_v, dst_v, rows_v, ex_v, ad_v,
                  acc_sh):
    cid = lax.axis_index("c")
    sid = lax.axis_index("s")
    if split_cores:
        wid = cid * NS + sid
    else:
        wid = sid

    heads_per_core = nheads if split_cores else nheads // NC
    nrow_e = EP // CHUNK

    for hh in range(heads_per_core):
        if split_cores:
            head = hh
        else:
            head = cid * heads_per_core + hh
        pltpu.sync_copy(zero_hbm.at[pl.ds(sid * ROWS_T, ROWS_T)],
                        acc_sh.at[pl.ds(sid * ROWS_T, ROWS_T)])
        pltpu.sync_copy(ad_hbm.at[pl.ds(head * NP, NP)], ad_v)
        plsc.subcore_barrier()

        iota16 = lax.iota(jnp.int32, 16)
        col_as = jnp.full((16,), C1 + 1, dtype=jnp.int32)

        def grp_body(grp, _, head=head):
            ebase = head * nrow_e + wid * nch + grp * GRP
            pltpu.sync_copy(src_hbm.at[pl.ds(ebase, GRP)], src_v)
            pltpu.sync_copy(dst_hbm.at[pl.ds(wid * nch + grp * GRP, GRP)],
                            dst_v)

            def chunk_body(ci, _):
                pltpu.sync_copy(h_hbm.at[src_v.at[ci]], rows_v)
                for g in range(CHUNK // 16):
                    r16 = iota16 + g * 16
                    asv = plsc.load_gather(rows_v, [r16, col_as])
                    d16 = dst_v[ci, pl.ds(g * 16, 16)]
                    e = asv + plsc.load_gather(ad_v, [d16])
                    e = jnp.where(e > 0, e, 0.2 * e)
                    ex_v[pl.ds(g * 16, 16)] = jnp.exp(e)

                def scale_body(k, _):
                    s = plsc.load_gather(
                        ex_v, [jnp.full((16,), k, dtype=jnp.int32)])
                    for j in range(FA // 16):
                        rows_v[k, pl.ds(j * 16, 16)] = (
                            rows_v[k, pl.ds(j * 16, 16)] * s)
                    return 0

                lax.fori_loop(0, CHUNK, scale_body, 0)
                pltpu.sync_copy(rows_v, acc_sh.at[dst_v.at[ci]], add=True)
                return 0

            lax.fori_loop(0, GRP, chunk_body, 0)
            return 0

        lax.fori_loop(0, nch // GRP, grp_body, 0)
        plsc.subcore_barrier()
        if split_cores:
            out_base = cid * NP
        else:
            out_base = head * NP
        pltpu.sync_copy(acc_sh.at[pl.ds(sid * ROWS_T, ROWS_T)],
                        num_hbm.at[pl.ds(out_base + sid * ROWS_T, ROWS_T)])


def _make_sc_kernel(nheads, nch, split_cores, out_leading):
    mesh = plsc.VectorSubcoreMesh(core_axis_name="c", subcore_axis_name="s",
                                  num_cores=NC, num_subcores=NS)
    return functools.partial(
        pl.kernel,
        out_type=jax.ShapeDtypeStruct((out_leading * NP, FA), jnp.float32),
        mesh=mesh,
        scratch_types=[
            pltpu.VMEM((GRP, CHUNK), jnp.int32),
            pltpu.VMEM((GRP, CHUNK), jnp.int32),
            pltpu.VMEM((CHUNK, FA), jnp.float32),
            pltpu.VMEM((CHUNK,), jnp.float32),
            pltpu.VMEM((NP,), jnp.float32),
            pltpu.VMEM_SHARED((NP, FA), jnp.float32),
        ],
        compiler_params=pltpu.CompilerParams(needs_layout_passes=False,
                                             use_tc_tiling_on_sc=False),
    )(functools.partial(_sc_edge_body, nheads, nch, split_cores))


_sc_layer1 = _make_sc_kernel(HEADS, NCH1, False, HEADS)
_sc_layer2 = _make_sc_kernel(1, NCH2, True, NC)


def _c_body(num_ref, w2_ref, b1_ref, as2_ref, ad2_ref, haug_ref, asad_ref):
    acc = jnp.zeros((256, C2), dtype=jnp.float32)
    for hh in range(HEADS):
        num = num_ref[hh, :, 0:C1]
        den = num_ref[hh, :, C1:C1 + 1]
        h1n = num / (den + 1e-16) + b1_ref[0, hh * C1:(hh + 1) * C1][None, :]
        h1n = jnp.maximum(h1n, 0.0)
        acc = acc + jnp.dot(h1n, w2_ref[hh * C1:(hh + 1) * C1, :],
                            preferred_element_type=jnp.float32)
    lane = lax.broadcasted_iota(jnp.int32, (256, 16), 1)
    ones_col = (lane == 0).astype(jnp.float32)
    as_col = (lane == 1).astype(jnp.float32)
    a_s = jnp.sum(acc * as2_ref[0, :][None, :], axis=1)
    haug_ref[:, 0:C2] = acc
    haug_ref[:, C2:FA] = ones_col + as_col * a_s[:, None]
    asad_ref[0, :] = jnp.sum(acc * ad2_ref[0, :][None, :], axis=1)
    asad_ref[1, :] = a_s


def _layer2_dense(num1, W2, b1, a_src2, a_dst2):
    return pl.pallas_call(
        _c_body,
        grid=(NBLK,),
        in_specs=[
            pl.BlockSpec((HEADS, 256, FA), lambda i: (0, i, 0)),
            pl.BlockSpec((HEADS * C1, C2), lambda i: (0, 0)),
            pl.BlockSpec((1, HEADS * C1), lambda i: (0, 0)),
            pl.BlockSpec((1, C2), lambda i: (0, 0)),
            pl.BlockSpec((1, C2), lambda i: (0, 0)),
        ],
        out_specs=[
            pl.BlockSpec((256, FA), lambda i: (i, 0)),
            pl.BlockSpec((2, 256), lambda i: (0, i)),
        ],
        out_shape=[
            jax.ShapeDtypeStruct((NP, FA), jnp.float32),
            jax.ShapeDtypeStruct((2, NP), jnp.float32),
        ],
    )(num1, W2, b1, a_src2, a_dst2)


def _d_body(num_ref, batch_ref, b2_ref, wa_ref, wm_ref, ba_ref, bm_ref,
            wo_ref, bo_ref, out_ref, pooled_ref):
    i = pl.program_id(0)
    num = num_ref[0, :, 0:C2] + num_ref[1, :, 0:C2]
    den = num_ref[0, :, C2:C2 + 1] + num_ref[1, :, C2:C2 + 1]
    h2 = jnp.maximum(num / (den + 1e-16) + b2_ref[0, :][None, :], 0.0)
    attn = jnp.sum(h2 * wa_ref[0, :][None, :], axis=1) + ba_ref[0, 0]
    gate = jax.nn.sigmoid(
        jnp.sum(h2 * wm_ref[0, :][None, :], axis=1) + bm_ref[0, 0])
    fs = attn * gate
    s_mat = h2 * fs[:, None]
    b = batch_ref[0, 0, :]
    oh = (lax.broadcasted_iota(jnp.int32, (G, 256), 0) == b[None, :]).astype(
        jnp.float32)
    contrib = jnp.dot(oh, s_mat, preferred_element_type=jnp.float32)

    @pl.when(i == 0)
    def _():
        pooled_ref[...] = contrib

    @pl.when(i > 0)
    def _():
        pooled_ref[...] = pooled_ref[...] + contrib

    @pl.when(i == NBLK - 1)
    def _():
        out_ref[...] = (
            jnp.dot(pooled_ref[...], wo_ref[...],
                    preferred_element_type=jnp.float32) + bo_ref[0, :][None, :])


def _pooling(num2, batch3, b2, wa, wm, ba, bm, Wo, bo):
    return pl.pallas_call(
        _d_body,
        grid=(NBLK,),
        in_specs=[
            pl.BlockSpec((NC, 256, FA), lambda i: (0, i, 0)),
            pl.BlockSpec((1, 1, 256), lambda i: (i, 0, 0)),
            pl.BlockSpec((1, C2), lambda i: (0, 0)),
            pl.BlockSpec((1, C2), lambda i: (0, 0)),
            pl.BlockSpec((1, C2), lambda i: (0, 0)),
            pl.BlockSpec((1, 1), lambda i: (0, 0)),
            pl.BlockSpec((1, 1), lambda i: (0, 0)),
            pl.BlockSpec((C2, OUT), lambda i: (0, 0)),
            pl.BlockSpec((1, OUT), lambda i: (0, 0)),
        ],
        out_specs=pl.BlockSpec((G, OUT), lambda i: (0, 0)),
        out_shape=jax.ShapeDtypeStruct((G, OUT), jnp.float32),
        scratch_shapes=[pltpu.VMEM((G, C2), jnp.float32)],
    )(num2, batch3, b2, wa, wm, ba, bm, Wo, bo)


def kernel(x, edge_index, batch, W1, a_src1, a_dst1, b1, W2, a_src2, a_dst2,
           b2, wa, ba, wm, bm, Wo, bo):
    x_pad = jnp.pad(x, ((0, NP - N), (0, 0)))
    src_p = jnp.pad(edge_index[0], (0, EP - E), constant_values=DUMMY)
    dst_p = jnp.pad(edge_index[1], (0, EP - E), constant_values=DUMMY)
    src1_r = (jnp.arange(HEADS, dtype=jnp.int32)[:, None] * NP +
              src_p[None, :]).reshape(HEADS * (EP // CHUNK), CHUNK)
    src2_r = src_p.reshape(EP // CHUNK, CHUNK)
    dst_r = dst_p.reshape(EP // CHUNK, CHUNK)
    batch3 = jnp.pad(batch, (0, NP - N),
                     constant_values=G).reshape(NBLK, 1, 256).astype(jnp.int32)
    zeros_nf = jnp.zeros((NP, FA), dtype=jnp.float32)

    haug1, ad1 = _layer1_dense(x_pad, W1, a_src1, a_dst1)
    num1 = _sc_layer1(haug1.reshape(HEADS * NP, FA),
                      ad1.reshape(HEADS * NP), src1_r, dst_r,
                      zeros_nf).reshape(HEADS, NP, FA)
    haug2, asad2 = _layer2_dense(num1, W2, b1.reshape(1, HEADS * C1),
                                 a_src2, a_dst2)
    num2 = _sc_layer2(haug2, asad2.reshape(2 * NP), src2_r, dst_r,
                      zeros_nf).reshape(NC, NP, FA)
    return _pooling(num2, batch3, b2.reshape(1, C2),
                    wa.reshape(1, C2), wm.reshape(1, C2),
                    ba.reshape(1, 1), bm.reshape(1, 1),
                    Wo, bo.reshape(1, OUT))


# speedup vs baseline: 11.2114x; 1.0975x over previous
import functools

import jax
import jax.numpy as jnp
from jax import lax
from jax.experimental import pallas as pl
from jax.experimental.pallas import tpu as pltpu
from jax.experimental.pallas import tpu_sc as plsc

N = 10000
NP = 10240
E = 320000
EP = 327680
FEAT = 128
HEADS = 8
C1 = 128
C2 = 128
OUT = 10
G = 64
FA = 144
DUMMY = N

NBLK = NP // 256

NC = 2
NS = 16
CHUNK = 128

EP_T1 = EP // NS
NCH1 = EP_T1 // CHUNK
EP_T2 = EP // (NS * NC)
NCH2 = EP_T2 // CHUNK
ROWS_T = NP // NS


def _a_body(x_ref, w_ref, asrc_ref, adst_ref, haug_ref, ad_ref):
    h = jnp.dot(x_ref[...], w_ref[...], preferred_element_type=jnp.float32)
    lane = lax.broadcasted_iota(jnp.int32, (256, 16), 1)
    ones_col = (lane == 0).astype(jnp.float32)
    as_col = (lane == 1).astype(jnp.float32)
    for hh in range(HEADS):
        hblk = h[:, hh * C1:(hh + 1) * C1]
        a_s = jnp.sum(hblk * asrc_ref[hh, :][None, :], axis=1)
        haug_ref[hh, :, 0:C1] = hblk
        haug_ref[hh, :, C1:FA] = ones_col + as_col * a_s[:, None]
        ad_ref[hh, :] = jnp.sum(hblk * adst_ref[hh, :][None, :], axis=1)


def _layer1_dense(x_pad, W1, a_src1, a_dst1):
    return pl.pallas_call(
        _a_body,
        grid=(NBLK,),
        in_specs=[
            pl.BlockSpec((256, FEAT), lambda i: (i, 0)),
            pl.BlockSpec((FEAT, HEADS * C1), lambda i: (0, 0)),
            pl.BlockSpec((HEADS, C1), lambda i: (0, 0)),
            pl.BlockSpec((HEADS, C1), lambda i: (0, 0)),
        ],
        out_specs=[
            pl.BlockSpec((HEADS, 256, FA), lambda i: (0, i, 0)),
            pl.BlockSpec((HEADS, 256), lambda i: (0, i)),
        ],
        out_shape=[
            jax.ShapeDtypeStruct((HEADS, NP, FA), jnp.float32),
            jax.ShapeDtypeStruct((HEADS, NP), jnp.float32),
        ],
    )(x_pad, W1, a_src1, a_dst1)


GRP = 16


def _sc_edge_body(nheads, nch, split_cores, h_hbm, ad_hbm, src_hbm, dst_hbm,
                  zero_hbm, num_hbm, src_v, dst_v, rows_v, ex_v, ad_v,
                  acc_sh):
    cid = lax.axis_index("c")
    sid = lax.axis_index("s")
    if split_cores:
        wid = cid * NS + sid
    else:
        wid = sid

    heads_per_core = nheads if split_cores else nheads // NC
    nrow_e = EP // CHUNK

    iota16 = lax.iota(jnp.int32, 16)
    col_as = jnp.full((16,), C1 + 1, dtype=jnp.int32)

    def head_body(hh, _):
        if split_cores:
            head = hh
        else:
            head = cid * heads_per_core + hh
        pltpu.sync_copy(zero_hbm.at[pl.ds(sid * ROWS_T, ROWS_T)],
                        acc_sh.at[pl.ds(sid * ROWS_T, ROWS_T)])
        pltpu.sync_copy(ad_hbm.at[pl.ds(head * NP, NP)], ad_v)
        plsc.subcore_barrier()

        def grp_body(grp, _):
            ebase = head * nrow_e + wid * nch + grp * GRP
            pltpu.sync_copy(src_hbm.at[pl.ds(ebase, GRP)], src_v)
            pltpu.sync_copy(dst_hbm.at[pl.ds(wid * nch + grp * GRP, GRP)],
                            dst_v)

            def chunk_body(ci, _):
                pltpu.sync_copy(h_hbm.at[src_v.at[ci]], rows_v)

                def ex_grp_body(g, _):
                    r16 = iota16 + g * 16
                    asv = plsc.load_gather(rows_v, [r16, col_as])
                    d16 = dst_v[ci, pl.ds(g * 16, 16)]
                    e = asv + plsc.load_gather(ad_v, [d16])
                    e = jnp.where(e > 0, e, 0.2 * e)
                    ex16 = jnp.exp(e)
                    for l in range(16):
                        s = jnp.full((16,), ex16[l], dtype=jnp.float32)
                        k = g * 16 + l
                        for j in range(FA // 16):
                            rows_v[k, pl.ds(j * 16, 16)] = (
                                rows_v[k, pl.ds(j * 16, 16)] * s)
                    return 0

                lax.fori_loop(0, CHUNK // 16, ex_grp_body, 0)
                pltpu.sync_copy(rows_v, acc_sh.at[dst_v.at[ci]], add=True)
                return 0

            lax.fori_loop(0, GRP, chunk_body, 0)
            return 0

        lax.fori_loop(0, nch // GRP, grp_body, 0)
        plsc.subcore_barrier()
        if split_cores:
            out_base = cid * NP
        else:
            out_base = head * NP
        pltpu.sync_copy(acc_sh.at[pl.ds(sid * ROWS_T, ROWS_T)],
                        num_hbm.at[pl.ds(out_base + sid * ROWS_T, ROWS_T)])
        return 0

    lax.fori_loop(0, heads_per_core, head_body, 0)


def _make_sc_kernel(nheads, nch, split_cores, out_leading):
    mesh = plsc.VectorSubcoreMesh(core_axis_name="c", subcore_axis_name="s",
                                  num_cores=NC, num_subcores=NS)
    return functools.partial(
        pl.kernel,
        out_type=jax.ShapeDtypeStruct((out_leading * NP, FA), jnp.float32),
        mesh=mesh,
        scratch_types=[
            pltpu.VMEM((GRP, CHUNK), jnp.int32),
            pltpu.VMEM((GRP, CHUNK), jnp.int32),
            pltpu.VMEM((CHUNK, FA), jnp.float32),
            pltpu.VMEM((16,), jnp.float32),
            pltpu.VMEM((NP,), jnp.float32),
            pltpu.VMEM_SHARED((NP, FA), jnp.float32),
        ],
        compiler_params=pltpu.CompilerParams(needs_layout_passes=False,
                                             use_tc_tiling_on_sc=False),
    )(functools.partial(_sc_edge_body, nheads, nch, split_cores))


_sc_layer1 = _make_sc_kernel(HEADS, NCH1, False, HEADS)
_sc_layer2 = _make_sc_kernel(1, NCH2, True, NC)


def _c_body(num_ref, w2_ref, b1_ref, as2_ref, ad2_ref, haug_ref, asad_ref):
    acc = jnp.zeros((256, C2), dtype=jnp.float32)
    for hh in range(HEADS):
        num = num_ref[hh, :, 0:C1]
        den = num_ref[hh, :, C1:C1 + 1]
        h1n = num / (den + 1e-16) + b1_ref[0, hh * C1:(hh + 1) * C1][None, :]
        h1n = jnp.maximum(h1n, 0.0)
        acc = acc + jnp.dot(h1n, w2_ref[hh * C1:(hh + 1) * C1, :],
                            preferred_element_type=jnp.float32)
    lane = lax.broadcasted_iota(jnp.int32, (256, 16), 1)
    ones_col = (lane == 0).astype(jnp.float32)
    as_col = (lane == 1).astype(jnp.float32)
    a_s = jnp.sum(acc * as2_ref[0, :][None, :], axis=1)
    haug_ref[:, 0:C2] = acc
    haug_ref[:, C2:FA] = ones_col + as_col * a_s[:, None]
    asad_ref[0, :] = jnp.sum(acc * ad2_ref[0, :][None, :], axis=1)
    asad_ref[1, :] = a_s


def _layer2_dense(num1, W2, b1, a_src2, a_dst2):
    return pl.pallas_call(
        _c_body,
        grid=(NBLK,),
        in_specs=[
            pl.BlockSpec((HEADS, 256, FA), lambda i: (0, i, 0)),
            pl.BlockSpec((HEADS * C1, C2), lambda i: (0, 0)),
            pl.BlockSpec((1, HEADS * C1), lambda i: (0, 0)),
            pl.BlockSpec((1, C2), lambda i: (0, 0)),
            pl.BlockSpec((1, C2), lambda i: (0, 0)),
        ],
        out_specs=[
            pl.BlockSpec((256, FA), lambda i: (i, 0)),
            pl.BlockSpec((2, 256), lambda i: (0, i)),
        ],
        out_shape=[
            jax.ShapeDtypeStruct((NP, FA), jnp.float32),
            jax.ShapeDtypeStruct((2, NP), jnp.float32),
        ],
    )(num1, W2, b1, a_src2, a_dst2)


def _d_body(num_ref, batch_ref, b2_ref, wa_ref, wm_ref, ba_ref, bm_ref,
            wo_ref, bo_ref, out_ref, pooled_ref):
    i = pl.program_id(0)
    num = num_ref[0, :, 0:C2] + num_ref[1, :, 0:C2]
    den = num_ref[0, :, C2:C2 + 1] + num_ref[1, :, C2:C2 + 1]
    h2 = jnp.maximum(num / (den + 1e-16) + b2_ref[0, :][None, :], 0.0)
    attn = jnp.sum(h2 * wa_ref[0, :][None, :], axis=1) + ba_ref[0, 0]
    gate = jax.nn.sigmoid(
        jnp.sum(h2 * wm_ref[0, :][None, :], axis=1) + bm_ref[0, 0])
    fs = attn * gate
    s_mat = h2 * fs[:, None]
    b = batch_ref[0, 0, :]
    oh = (lax.broadcasted_iota(jnp.int32, (G, 256), 0) == b[None, :]).astype(
        jnp.float32)
    contrib = jnp.dot(oh, s_mat, preferred_element_type=jnp.float32)

    @pl.when(i == 0)
    def _():
        pooled_ref[...] = contrib

    @pl.when(i > 0)
    def _():
        pooled_ref[...] = pooled_ref[...] + contrib

    @pl.when(i == NBLK - 1)
    def _():
        out_ref[...] = (
            jnp.dot(pooled_ref[...], wo_ref[...],
                    preferred_element_type=jnp.float32) + bo_ref[0, :][None, :])


def _pooling(num2, batch3, b2, wa, wm, ba, bm, Wo, bo):
    return pl.pallas_call(
        _d_body,
        grid=(NBLK,),
        in_specs=[
            pl.BlockSpec((NC, 256, FA), lambda i: (0, i, 0)),
            pl.BlockSpec((1, 1, 256), lambda i: (i, 0, 0)),
            pl.BlockSpec((1, C2), lambda i: (0, 0)),
            pl.BlockSpec((1, C2), lambda i: (0, 0)),
            pl.BlockSpec((1, C2), lambda i: (0, 0)),
            pl.BlockSpec((1, 1), lambda i: (0, 0)),
            pl.BlockSpec((1, 1), lambda i: (0, 0)),
            pl.BlockSpec((C2, OUT), lambda i: (0, 0)),
            pl.BlockSpec((1, OUT), lambda i: (0, 0)),
        ],
        out_specs=pl.BlockSpec((G, OUT), lambda i: (0, 0)),
        out_shape=jax.ShapeDtypeStruct((G, OUT), jnp.float32),
        scratch_shapes=[pltpu.VMEM((G, C2), jnp.float32)],
    )(num2, batch3, b2, wa, wm, ba, bm, Wo, bo)


def kernel(x, edge_index, batch, W1, a_src1, a_dst1, b1, W2, a_src2, a_dst2,
           b2, wa, ba, wm, bm, Wo, bo):
    x_pad = jnp.pad(x, ((0, NP - N), (0, 0)))
    src_p = jnp.pad(edge_index[0], (0, EP - E), constant_values=DUMMY)
    dst_p = jnp.pad(edge_index[1], (0, EP - E), constant_values=DUMMY)
    src1_r = (jnp.arange(HEADS, dtype=jnp.int32)[:, None] * NP +
              src_p[None, :]).reshape(HEADS * (EP // CHUNK), CHUNK)
    src2_r = src_p.reshape(EP // CHUNK, CHUNK)
    dst_r = dst_p.reshape(EP // CHUNK, CHUNK)
    batch3 = jnp.pad(batch, (0, NP - N),
                     constant_values=G).reshape(NBLK, 1, 256).astype(jnp.int32)
    zeros_nf = jnp.zeros((NP, FA), dtype=jnp.float32)

    haug1, ad1 = _layer1_dense(x_pad, W1, a_src1, a_dst1)
    num1 = _sc_layer1(haug1.reshape(HEADS * NP, FA),
                      ad1.reshape(HEADS * NP), src1_r, dst_r,
                      zeros_nf).reshape(HEADS, NP, FA)
    haug2, asad2 = _layer2_dense(num1, W2, b1.reshape(1, HEADS * C1),
                                 a_src2, a_dst2)
    num2 = _sc_layer2(haug2, asad2.reshape(2 * NP), src2_r, dst_r,
                      zeros_nf).reshape(NC, NP, FA)
    return _pooling(num2, batch3, b2.reshape(1, C2),
                    wa.reshape(1, C2), wm.reshape(1, C2),
                    ba.reshape(1, 1), bm.reshape(1, 1),
                    Wo, bo.reshape(1, OUT))
